# Initial kernel scaffold; baseline (speedup 1.0000x reference)
import jax
import jax.numpy as jnp
from jax.experimental import pallas as pl


def kernel(heatmaps, center, reproLookup):
    raise NotImplementedError("write your pallas kernel here")


# speedup vs baseline: 3.1160x; 3.1160x over previous
import functools

import jax
import jax.numpy as jnp
from jax import lax
from jax.experimental import pallas as pl
from jax.experimental.pallas import tpu as pltpu
from jax.experimental.pallas import tpu_sc as plsc

B = 2
C = 12
J = 23
JP = 32
H, W = 128, 640
HW = H * W
G = 52
G3 = G * G * G
GRID_SPACING = 2.0
OFFSET = -100.0

NC, NS = 2, 16
NW = NC * NS
RPB = 128
BPT = 35
G3P = NW * BPT * RPB
INV_C = 1.0 / C

_mesh = plsc.VectorSubcoreMesh(core_axis_name="c", subcore_axis_name="s")


@functools.partial(
    pl.kernel,
    out_type=jax.ShapeDtypeStruct((B, G3P, JP), jnp.float32),
    mesh=_mesh,
    scratch_types=[
        pltpu.VMEM((C, RPB), jnp.int32),
        pltpu.VMEM((C, RPB, JP), jnp.float32),
        pltpu.VMEM((RPB, JP), jnp.float32),
        pltpu.SemaphoreType.DMA,
    ],
    compiler_params=pltpu.CompilerParams(use_tc_tiling_on_sc=False),
)
def _sc_reproject(table_hbm, idx_hbm, out_hbm, idx_v, rows_v, out_v, sem):
    wid = lax.axis_index("s") * NC + lax.axis_index("c")

    for b in range(B):
        def blk_body(blk, _, b=b):
            base = (wid * BPT + blk) * RPB
            pltpu.sync_copy(idx_hbm.at[b, :, pl.ds(base, RPB)], idx_v)
            copies = [
                pltpu.async_copy(table_hbm.at[idx_v.at[cc]], rows_v.at[cc], sem)
                for cc in range(C)
            ]
            for cp in copies:
                cp.wait()

            def row_body(i, _):
                for h in range(2):
                    sl = pl.ds(h * 16, 16)
                    acc = rows_v[0, i, sl]
                    for cc in range(1, C):
                        acc = acc + rows_v[cc, i, sl]
                    out_v[i, sl] = acc * INV_C
                return 0

            lax.fori_loop(0, RPB, row_body, 0)
            pltpu.sync_copy(out_v, out_hbm.at[b, pl.ds(base, RPB), :])
            return 0

        lax.fori_loop(0, BPT, blk_body, 0)


def kernel(heatmaps, center, reproLookup):
    ci = ((center - OFFSET) / GRID_SPACING).astype(jnp.int32)
    crops = []
    for b in range(B):
        start = (jnp.int32(0), ci[b, 0] - G // 2, ci[b, 1] - G // 2, ci[b, 2] - G // 2)
        crops.append(lax.dynamic_slice(reproLookup, start, (C, G, G, G)))
    idx = jnp.stack(crops).reshape(B, C, G3)
    offs = (jnp.arange(B, dtype=jnp.int32)[:, None] * C
            + jnp.arange(C, dtype=jnp.int32)[None, :]) * HW
    idx = idx + offs[:, :, None]
    idx = jnp.pad(idx, ((0, 0), (0, 0), (0, G3P - G3)))
    hm_t = jnp.transpose(heatmaps.reshape(B, C, J, HW), (0, 1, 3, 2))
    hm_t = jnp.pad(hm_t, ((0, 0), (0, 0), (0, 0), (0, JP - J)))
    table = hm_t.reshape(B * C * HW, JP)

    out = _sc_reproject(table, idx)
    out = out[:, :G3, :J]
    return jnp.transpose(out, (0, 2, 1)).reshape(B, J, G, G, G)
